# baseline (speedup 1.0000x reference)
import functools

import jax
import jax.numpy as jnp
from jax import lax
from jax.experimental import pallas as pl
from jax.experimental.pallas import tpu as pltpu
from jax.experimental.pallas import tpu_sc as plsc

D_MODEL = 1024
NUM_CORES = 2
NUM_SUBCORES = 16
NW = NUM_CORES * NUM_SUBCORES

K = 48
NBUF = 2


def _make_gather(n_idx):
    per_w = n_idx // NW
    chunks = []
    off = 0
    while off < per_w:
        c = min(K, per_w - off)
        chunks.append((off, c))
        off += c
    nch = len(chunks)
    mesh = plsc.VectorSubcoreMesh(core_axis_name="c", subcore_axis_name="s")

    @functools.partial(
        pl.kernel,
        mesh=mesh,
        out_type=jax.ShapeDtypeStruct((n_idx, D_MODEL), jnp.float32),
        scratch_types=[
            pltpu.VMEM((per_w,), jnp.int32),
            [pltpu.VMEM((K, D_MODEL), jnp.float32) for _ in range(NBUF)],
            [pltpu.SemaphoreType.DMA for _ in range(NBUF)],
            [pltpu.SemaphoreType.DMA for _ in range(NBUF)],
        ],
    )
    def gather_kernel(idx_hbm, table_hbm, out_hbm, idx_v, bufs, gsems, ssems):
        wid = lax.axis_index("s") * NUM_CORES + lax.axis_index("c")
        base = wid * per_w
        pltpu.sync_copy(idx_hbm.at[pl.ds(base, per_w)], idx_v)

        def gath_start(g):
            p = g % NBUF
            off, c = chunks[g]

            def issue(j, carry):
                vec = idx_v[pl.ds(off + j * 16, 16)]
                for lane in range(16):
                    pltpu.make_async_copy(
                        table_hbm.at[pl.ds(vec[lane], 1)],
                        bufs[p].at[pl.ds(j * 16 + lane, 1)],
                        gsems[p],
                    ).start()
                return carry

            lax.fori_loop(0, c // 16, issue, 0)

        def gath_wait(g):
            p = g % NBUF
            off, c = chunks[g]
            dst = bufs[p] if c == K else bufs[p].at[pl.ds(0, c)]
            pltpu.make_async_copy(table_hbm.at[pl.ds(0, c)], dst, gsems[p]).wait()

        def scat(g):
            p = g % NBUF
            off, c = chunks[g]
            src = bufs[p] if c == K else bufs[p].at[pl.ds(0, c)]
            return pltpu.make_async_copy(
                src, out_hbm.at[pl.ds(base + off, c)], ssems[p]
            )

        for g in range(min(NBUF - 1, nch)):
            gath_start(g)
        for g in range(nch):
            gath_wait(g)
            scat(g).start()
            nxt = g + NBUF - 1
            if nxt < nch:
                if nxt - NBUF >= 0:
                    scat(nxt - NBUF).wait()
                gath_start(nxt)
        for g in range(max(nch - NBUF, 0), nch):
            scat(g).wait()

    return gather_kernel


def kernel(timesteps, pe, index_select):
    del index_select
    b, l = timesteps.shape
    flat = timesteps.reshape(-1).astype(jnp.int32)
    out = _make_gather(flat.shape[0])(flat, pe)
    return out.reshape(b, l, pe.shape[-1])


# speedup vs baseline: 1.0190x; 1.0190x over previous
import functools

import jax
import jax.numpy as jnp
from jax import lax
from jax.experimental import pallas as pl
from jax.experimental.pallas import tpu as pltpu
from jax.experimental.pallas import tpu_sc as plsc

D_MODEL = 1024
NUM_CORES = 2
NUM_SUBCORES = 16
NW = NUM_CORES * NUM_SUBCORES

K = 16
NBUF = 6


def _make_gather(n_idx):
    per_w = n_idx // NW
    chunks = []
    off = 0
    while off < per_w:
        c = min(K, per_w - off)
        chunks.append((off, c))
        off += c
    nch = len(chunks)
    mesh = plsc.VectorSubcoreMesh(core_axis_name="c", subcore_axis_name="s")

    @functools.partial(
        pl.kernel,
        mesh=mesh,
        out_type=jax.ShapeDtypeStruct((n_idx, D_MODEL), jnp.float32),
        scratch_types=[
            pltpu.VMEM((per_w,), jnp.int32),
            [pltpu.VMEM((K, D_MODEL), jnp.float32) for _ in range(NBUF)],
            [pltpu.SemaphoreType.DMA for _ in range(NBUF)],
            [pltpu.SemaphoreType.DMA for _ in range(NBUF)],
        ],
    )
    def gather_kernel(idx_hbm, table_hbm, out_hbm, idx_v, bufs, gsems, ssems):
        wid = lax.axis_index("s") * NUM_CORES + lax.axis_index("c")
        base = wid * per_w
        pltpu.sync_copy(idx_hbm.at[pl.ds(base, per_w)], idx_v)

        def gath(g):
            p = g % NBUF
            off, c = chunks[g]
            dst = bufs[p] if c == K else bufs[p].at[pl.ds(0, c)]
            return pltpu.make_async_copy(
                table_hbm.at[idx_v.at[pl.ds(off, c)]], dst, gsems[p]
            )

        def gath_start(g):
            p = g % NBUF
            off, c = chunks[g]
            if g % 2 == 0:
                gath(g).start()
                return

            def issue(j, carry):
                vec = idx_v[pl.ds(off + j * 16, 16)]
                for lane in range(16):
                    pltpu.make_async_copy(
                        table_hbm.at[pl.ds(vec[lane], 1)],
                        bufs[p].at[pl.ds(j * 16 + lane, 1)],
                        gsems[p],
                    ).start()
                return carry

            lax.fori_loop(0, c // 16, issue, 0)

        def scat(g):
            p = g % NBUF
            off, c = chunks[g]
            src = bufs[p] if c == K else bufs[p].at[pl.ds(0, c)]
            return pltpu.make_async_copy(
                src, out_hbm.at[pl.ds(base + off, c)], ssems[p]
            )

        for g in range(min(NBUF - 1, nch)):
            gath_start(g)
        for g in range(nch):
            gath(g).wait()
            scat(g).start()
            nxt = g + NBUF - 1
            if nxt < nch:
                if nxt - NBUF >= 0:
                    scat(nxt - NBUF).wait()
                gath_start(nxt)
        for g in range(max(nch - NBUF, 0), nch):
            scat(g).wait()

    return gather_kernel


def kernel(timesteps, pe, index_select):
    del index_select
    b, l = timesteps.shape
    flat = timesteps.reshape(-1).astype(jnp.int32)
    out = _make_gather(flat.shape[0])(flat, pe)
    return out.reshape(b, l, pe.shape[-1])


# speedup vs baseline: 1.0831x; 1.0629x over previous
import functools

import jax
import jax.numpy as jnp
from jax import lax
from jax.experimental import pallas as pl
from jax.experimental.pallas import tpu as pltpu
from jax.experimental.pallas import tpu_sc as plsc

D_MODEL = 1024
NUM_CORES = 2
NUM_SUBCORES = 16
NW = NUM_CORES * NUM_SUBCORES

K = 16
NBUF = 6


def _make_gather(n_idx):
    per_w = n_idx // NW
    chunks = []
    off = 0
    while off < per_w:
        c = min(K, per_w - off)
        chunks.append((off, c))
        off += c
    nch = len(chunks)
    mesh = plsc.VectorSubcoreMesh(core_axis_name="c", subcore_axis_name="s")

    @functools.partial(
        pl.kernel,
        mesh=mesh,
        out_type=jax.ShapeDtypeStruct((n_idx, D_MODEL), jnp.float32),
        scratch_types=[
            pltpu.VMEM((per_w,), jnp.int32),
            [pltpu.VMEM((K, D_MODEL), jnp.float32) for _ in range(NBUF)],
            [pltpu.SemaphoreType.DMA for _ in range(NBUF)],
            [pltpu.SemaphoreType.DMA for _ in range(NBUF)],
        ],
    )
    def gather_kernel(idx_hbm, table_hbm, out_hbm, idx_v, bufs, gsems, ssems):
        wid = lax.axis_index("s") * NUM_CORES + lax.axis_index("c")
        base = wid * per_w
        pltpu.sync_copy(idx_hbm.at[pl.ds(base, per_w)], idx_v)

        def gath(g):
            p = g % NBUF
            off, c = chunks[g]
            dst = bufs[p] if c == K else bufs[p].at[pl.ds(0, c)]
            return pltpu.make_async_copy(
                table_hbm.at[idx_v.at[pl.ds(off, c)]], dst, gsems[p]
            )

        def scat(g):
            p = g % NBUF
            off, c = chunks[g]
            src = bufs[p] if c == K else bufs[p].at[pl.ds(0, c)]
            return pltpu.make_async_copy(
                src, out_hbm.at[pl.ds(base + off, c)], ssems[p]
            )

        for g in range(min(NBUF - 1, nch)):
            gath(g).start()
        for g in range(nch):
            gath(g).wait()
            scat(g).start()
            nxt = g + NBUF - 1
            if nxt < nch:
                if nxt - NBUF >= 0:
                    scat(nxt - NBUF).wait()
                gath(nxt).start()
        for g in range(max(nch - NBUF, 0), nch):
            scat(g).wait()

    return gather_kernel


def kernel(timesteps, pe, index_select):
    del index_select
    b, l = timesteps.shape
    flat = timesteps.reshape(-1).astype(jnp.int32)
    out = _make_gather(flat.shape[0])(flat, pe)
    return out.reshape(b, l, pe.shape[-1])
